# speedup vs baseline: 3.3568x; 3.3568x over previous
import functools

import jax
import jax.numpy as jnp
from jax import lax
from jax.experimental import pallas as pl
from jax.experimental.pallas import tpu as pltpu
from jax.experimental.pallas import tpu_sc as plsc

NC = 2
NS = 16
NW = NC * NS
D = 128
CH = 128
NBUF = 5


def _build(n_rows):
    nch_w = n_rows // (NW * CH)
    assert nch_w * NW * CH == n_rows and nch_w % NBUF == 0

    mesh = plsc.VectorSubcoreMesh(core_axis_name="c", subcore_axis_name="s")

    @functools.partial(
        pl.kernel,
        out_type=jax.ShapeDtypeStruct((n_rows, D), jnp.float32),
        mesh=mesh,
        scratch_types=[
            pltpu.VMEM((nch_w, CH), jnp.int32),
            pltpu.VMEM((NBUF, CH, D), jnp.float32),
        ] + [pltpu.SemaphoreType.DMA] * NBUF,
    )
    def gather_kernel(idx_hbm, table_hbm, out_hbm, idx_v, rows_v, *sems):
        wid = lax.axis_index("s") * NC + lax.axis_index("c")
        pltpu.sync_copy(idx_hbm.at[wid], idx_v)
        row0 = wid * (nch_w * CH)

        def start(g, b):
            pltpu.async_copy(table_hbm.at[idx_v.at[g]], rows_v.at[b], sems[b])

        def finish(g, b):
            pltpu.make_async_copy(
                table_hbm.at[idx_v.at[g]], rows_v.at[b], sems[b]
            ).wait()

        for b in range(NBUF):
            start(b, b)

        def group(i, carry):
            g0 = i * NBUF
            for b in range(NBUF):
                g = g0 + b
                finish(g, b)
                pltpu.sync_copy(
                    rows_v.at[b], out_hbm.at[pl.ds(row0 + g * CH, CH)]
                )
                nxt = g + NBUF

                @pl.when(nxt < nch_w)
                def _():
                    start(nxt, b)
            return carry

        lax.fori_loop(0, nch_w // NBUF, group, None)

    return gather_kernel


@jax.jit
def kernel(x, table):
    b, s = x.shape
    n = b * s
    idx3 = x.reshape(NW, n // (NW * CH), CH)
    out = _build(n)(idx3, table)
    return out.reshape(b, s, D)


# speedup vs baseline: 5.9848x; 1.7829x over previous
import functools

import jax
import jax.numpy as jnp
from jax import lax
from jax.experimental import pallas as pl
from jax.experimental.pallas import tpu as pltpu
from jax.experimental.pallas import tpu_sc as plsc

NC = 2
NS = 16
NW = NC * NS
D = 128
NBUF = 8


def _build(b, s):
    rows_w = b // NW
    assert rows_w * NW == b and rows_w % NBUF == 0

    mesh = plsc.VectorSubcoreMesh(core_axis_name="c", subcore_axis_name="s")

    @functools.partial(
        pl.kernel,
        out_type=jax.ShapeDtypeStruct((b, s, D), jnp.float32),
        mesh=mesh,
        scratch_types=[
            pltpu.VMEM((rows_w, s), jnp.int32),
            pltpu.VMEM((NBUF, s, D), jnp.float32),
        ] + [pltpu.SemaphoreType.DMA] * NBUF,
    )
    def gather_kernel(idx_hbm, table_hbm, out_hbm, idx_v, rows_v, *sems):
        wid = lax.axis_index("s") * NC + lax.axis_index("c")
        row0 = wid * rows_w
        pltpu.sync_copy(idx_hbm.at[pl.ds(row0, rows_w)], idx_v)

        def start(g, buf):
            pltpu.async_copy(table_hbm.at[idx_v.at[g]], rows_v.at[buf], sems[buf])

        def finish(g, buf):
            pltpu.make_async_copy(
                table_hbm.at[idx_v.at[g]], rows_v.at[buf], sems[buf]
            ).wait()

        for buf in range(NBUF):
            start(buf, buf)

        def group(i, carry):
            g0 = i * NBUF
            for buf in range(NBUF):
                g = g0 + buf
                finish(g, buf)
                pltpu.sync_copy(rows_v.at[buf], out_hbm.at[row0 + g])
                nxt = g + NBUF

                @pl.when(nxt < rows_w)
                def _():
                    start(nxt, buf)
            return carry

        lax.fori_loop(0, rows_w // NBUF, group, None)

    return gather_kernel


@jax.jit
def kernel(x, table):
    b, s = x.shape
    return _build(b, s)(x, table)
